# baseline (speedup 1.0000x reference)
import functools

import jax
import jax.numpy as jnp
from jax import lax
from jax.experimental import pallas as pl
from jax.experimental.pallas import tpu as pltpu
from jax.experimental.pallas import tpu_sc as plsc

N = 10000
D = 128
E = 320000
R = 512

NC = 2
NS = 16
NW = NC * NS
EPW = E // NW
K = 80
NB = EPW // K
SB = 25
NSC = NB // SB
NPAD = 10240
ROWS_PER_SUB = NPAD // NS
RPW = R // NS

_mesh = plsc.VectorSubcoreMesh(core_axis_name="c", subcore_axis_name="s")


def _scale_rows(rows, w_v, base, bias_v=None):
    def grp(g, c):
        wv = w_v[pl.ds(base + g * 16, 16)]
        for r in range(16):
            lane = jnp.full((16, 1), r, dtype=jnp.int32)
            ws = lax.gather(
                wv, lane,
                lax.GatherDimensionNumbers(
                    offset_dims=(), collapsed_slice_dims=(0,),
                    start_index_map=(0,)),
                slice_sizes=(1,),
                mode=lax.GatherScatterMode.PROMISE_IN_BOUNDS)
            row = g * 16 + r
            for j in range(8):
                sl = pl.ds(j * 16, 16)
                v = rows[row, sl]
                if bias_v is not None:
                    v = jnp.maximum(v, 0.0) + bias_v[sl]
                rows[row, sl] = v * ws
        return c

    lax.fori_loop(0, K // 16, grp, 0)


def _sc_pass_body(src_hbm, gidx_hbm, sidx_hbm, w_hbm, part_out,
                  gidx_v, sidx_v, w_v, rows0, rows1, rows2, acc,
                  gsem0, gsem1, gsem2, ssem0, ssem1, ssem2, bias_v=None):
    cid = lax.axis_index("c")
    sid = lax.axis_index("s")
    wid = cid * NS + sid

    zv = jnp.zeros((16,), jnp.float32)

    def zrow(i, c):
        for j in range(8):
            rows0[i, pl.ds(j * 16, 16)] = zv
        return c

    lax.fori_loop(0, K, zrow, 0)
    zbase = sid * ROWS_PER_SUB
    for t in range(ROWS_PER_SUB // K):
        pltpu.sync_copy(rows0, acc.at[pl.ds(zbase + t * K, K)])
    plsc.subcore_barrier()

    def third(bl, cur, gsem_c, ssem_c, prv, gsem_p, ssem_p,
              first=False, last=False, guard=False):
        if first:
            @pl.when(bl > 0)
            def _():
                pltpu.make_async_copy(
                    prv, acc.at[sidx_v.at[bl - 1]], ssem_p).wait()
                pltpu.async_copy(src_hbm.at[gidx_v.at[bl + 2]], prv, gsem_p)

            @pl.when(bl == 0)
            def _():
                pltpu.async_copy(src_hbm.at[gidx_v.at[bl + 2]], prv, gsem_p)
        elif last:
            pltpu.make_async_copy(
                prv, acc.at[sidx_v.at[bl - 1]], ssem_p).wait()
        elif guard:
            pltpu.make_async_copy(
                prv, acc.at[sidx_v.at[bl - 1]], ssem_p).wait()

            @pl.when(bl + 2 < SB)
            def _():
                pltpu.async_copy(src_hbm.at[gidx_v.at[bl + 2]], prv, gsem_p)
        else:
            pltpu.make_async_copy(
                prv, acc.at[sidx_v.at[bl - 1]], ssem_p).wait()
            pltpu.async_copy(src_hbm.at[gidx_v.at[bl + 2]], prv, gsem_p)
        pltpu.make_async_copy(src_hbm.at[gidx_v.at[bl]], cur, gsem_c).wait()
        _scale_rows(cur, w_v, bl * K, bias_v)
        pltpu.async_copy(cur, acc.at[sidx_v.at[bl]], ssem_c, add=True)

    def chunk(s, carry):
        pltpu.sync_copy(gidx_hbm.at[wid, s], gidx_v)
        pltpu.sync_copy(sidx_hbm.at[wid, s], sidx_v)
        pltpu.sync_copy(w_hbm.at[pl.ds(wid * EPW + s * (SB * K), SB * K)],
                        w_v)
        pltpu.async_copy(src_hbm.at[gidx_v.at[0]], rows0, gsem0)
        pltpu.async_copy(src_hbm.at[gidx_v.at[1]], rows1, gsem1)

        def triple(t, c2):
            third(3 * t, rows0, gsem0, ssem0, rows2, gsem2, ssem2,
                  first=True)
            third(3 * t + 1, rows1, gsem1, ssem1, rows0, gsem0, ssem0)
            third(3 * t + 2, rows2, gsem2, ssem2, rows1, gsem1, ssem1,
                  guard=True)
            return c2

        lax.fori_loop(0, SB // 3, triple, 0)
        third(SB - 1, rows0, gsem0, ssem0, rows2, gsem2, ssem2, last=True)
        pltpu.make_async_copy(
            rows0, acc.at[sidx_v.at[SB - 1]], ssem0).wait()
        return carry

    lax.fori_loop(0, NSC, chunk, 0)
    plsc.subcore_barrier()

    pltpu.sync_copy(acc.at[pl.ds(zbase, ROWS_PER_SUB)],
                    part_out.at[cid, pl.ds(zbase, ROWS_PER_SUB)])


@functools.partial(
    pl.kernel,
    mesh=_mesh,
    out_type=[jax.ShapeDtypeStruct((2, NPAD, D), jnp.float32),
              jax.ShapeDtypeStruct((R, D), jnp.float32)],
    scratch_types=[
        pltpu.VMEM((SB, K), jnp.int32),
        pltpu.VMEM((SB, K), jnp.int32),
        pltpu.VMEM((SB * K,), jnp.float32),
        pltpu.VMEM((K, D), jnp.float32),
        pltpu.VMEM((K, D), jnp.float32),
        pltpu.VMEM((K, D), jnp.float32),
        pltpu.VMEM_SHARED((NPAD, D), jnp.float32),
        pltpu.SemaphoreType.DMA,
        pltpu.SemaphoreType.DMA,
        pltpu.SemaphoreType.DMA,
        pltpu.SemaphoreType.DMA,
        pltpu.SemaphoreType.DMA,
        pltpu.SemaphoreType.DMA,
        pltpu.VMEM((RPW,), jnp.int32),
        pltpu.VMEM((D,), jnp.float32),
        pltpu.SemaphoreType.DMA,
    ],
)
def _sc_pass1(x_hbm, gidx_hbm, sidx_hbm, w_hbm, bias_hbm, ridx_hbm,
              part_out, root_out,
              gidx_v, sidx_v, w_v, rows0, rows1, rows2, acc,
              gsem0, gsem1, gsem2, ssem0, ssem1, ssem2, ridx_v, bias_v,
              rsem):
    cid = lax.axis_index("c")
    sid = lax.axis_index("s")
    pltpu.sync_copy(bias_hbm, bias_v)

    @pl.when(cid == 0)
    def _root():
        pltpu.sync_copy(ridx_hbm.at[pl.ds(sid * RPW, RPW)], ridx_v)
        rdst = rows0.at[pl.ds(0, RPW)]
        pltpu.async_copy(x_hbm.at[ridx_v], rdst, rsem).wait()
        pltpu.sync_copy(rdst, root_out.at[pl.ds(sid * RPW, RPW)])

    _sc_pass_body(x_hbm, gidx_hbm, sidx_hbm, w_hbm, part_out,
                  gidx_v, sidx_v, w_v, rows0, rows1, rows2, acc,
                  gsem0, gsem1, gsem2, ssem0, ssem1, ssem2, bias_v)


@functools.partial(
    pl.kernel,
    mesh=_mesh,
    out_type=[jax.ShapeDtypeStruct((2, NPAD, D), jnp.float32)],
    scratch_types=[
        pltpu.VMEM((SB, K), jnp.int32),
        pltpu.VMEM((SB, K), jnp.int32),
        pltpu.VMEM((SB * K,), jnp.float32),
        pltpu.VMEM((K, D), jnp.float32),
        pltpu.VMEM((K, D), jnp.float32),
        pltpu.VMEM((K, D), jnp.float32),
        pltpu.VMEM_SHARED((NPAD, D), jnp.float32),
        pltpu.SemaphoreType.DMA,
        pltpu.SemaphoreType.DMA,
        pltpu.SemaphoreType.DMA,
        pltpu.SemaphoreType.DMA,
        pltpu.SemaphoreType.DMA,
        pltpu.SemaphoreType.DMA,
    ],
)
def _sc_pass2(src_hbm, gidx_hbm, sidx_hbm, w_hbm, part_out,
              gidx_v, sidx_v, w_v, rows0, rows1, rows2, acc,
              gsem0, gsem1, gsem2, ssem0, ssem1, ssem2):
    _sc_pass_body(src_hbm, gidx_hbm, sidx_hbm, w_hbm, part_out,
                  gidx_v, sidx_v, w_v, rows0, rows1, rows2, acc,
                  gsem0, gsem1, gsem2, ssem0, ssem1, ssem2)


_BLK = 400
_GRID = N // _BLK


def _sum_partials(part):
    def body(a_ref, b_ref, o_ref):
        o_ref[...] = a_ref[0] + b_ref[0]

    return pl.pallas_call(
        body,
        grid=(_GRID,),
        in_specs=[pl.BlockSpec((1, _BLK, D), lambda i: (0, i, 0)),
                  pl.BlockSpec((1, _BLK, D), lambda i: (1, i, 0))],
        out_specs=pl.BlockSpec((_BLK, D), lambda i: (i, 0)),
        out_shape=jax.ShapeDtypeStruct((N, D), jnp.float32),
    )(part, part)


def _combine(part, x, fc1_Wt, W1t, b1, w2, b2):
    def body(p0_ref, p1_ref, x_ref, fw_ref, w1_ref, b1_ref, w2_ref, b2_ref,
             o_ref):
        node = p0_ref[0] + p1_ref[0]
        node = node - jnp.max(node, axis=1, keepdims=True)
        ex = jnp.exp(node)
        sm = ex / jnp.sum(ex, axis=1, keepdims=True)
        node2 = jnp.dot(sm, fw_ref[...], preferred_element_type=jnp.float32)
        xb = x_ref[...]
        b1v = b1_ref[...]
        tx = jnp.tanh(jnp.dot(xb, w1_ref[...],
                              preferred_element_type=jnp.float32) + b1v)
        tn = jnp.tanh(jnp.dot(node2, w1_ref[...],
                              preferred_element_type=jnp.float32) + b1v)
        w2v = w2_ref[...]
        b2v = b2_ref[0, 0]
        sx = jnp.sum(tx * w2v, axis=1, keepdims=True) + b2v
        sn = jnp.sum(tn * w2v, axis=1, keepdims=True) + b2v
        m = jnp.maximum(sx, sn)
        ax = jnp.exp(sx - m)
        an = jnp.exp(sn - m)
        o_ref[...] = (ax * xb + an * node2) / (ax + an)

    return pl.pallas_call(
        body,
        grid=(_GRID,),
        in_specs=[pl.BlockSpec((1, _BLK, D), lambda i: (0, i, 0)),
                  pl.BlockSpec((1, _BLK, D), lambda i: (1, i, 0)),
                  pl.BlockSpec((_BLK, D), lambda i: (i, 0)),
                  pl.BlockSpec((D, D), lambda i: (0, 0)),
                  pl.BlockSpec((D, D), lambda i: (0, 0)),
                  pl.BlockSpec((1, D), lambda i: (0, 0)),
                  pl.BlockSpec((1, D), lambda i: (0, 0)),
                  pl.BlockSpec((1, 1), lambda i: (0, 0))],
        out_specs=pl.BlockSpec((_BLK, D), lambda i: (i, 0)),
        out_shape=jax.ShapeDtypeStruct((N, D), jnp.float32),
    )(part, part, x, fc1_Wt, W1t, b1, w2, b2)


def kernel(x, edge_index, edge_weight, root_idx, hgc1_bias, fc1_W,
           fus_l1_W, fus_l1_b, fus_l2_W, fus_l2_b):
    row = edge_index[0].reshape(NW, NSC, SB, K)
    col = edge_index[1].reshape(NW, NSC, SB, K)

    part1, root_emb = _sc_pass1(x, row, col, edge_weight, hgc1_bias,
                                root_idx)
    edge = _sum_partials(part1)
    (part2,) = _sc_pass2(edge, col, row, edge_weight)
    out = _combine(part2, x, fc1_W.T, fus_l1_W.T,
                   fus_l1_b.reshape(1, D), fus_l2_W.reshape(1, D),
                   fus_l2_b.reshape(1, 1))
    return (out, edge, root_emb)


# speedup vs baseline: 2.1212x; 2.1212x over previous
import functools

import jax
import jax.numpy as jnp
from jax import lax
from jax.experimental import pallas as pl
from jax.experimental.pallas import tpu as pltpu
from jax.experimental.pallas import tpu_sc as plsc

N = 10000
D = 128
E = 320000
R = 512

NC = 2
NS = 16
NW = NC * NS
EPW = E // NW
K = 80
NB = EPW // K
SB = 25
NSC = NB // SB
NPAD = 10240
ROWS_PER_SUB = NPAD // NS
RPW = R // NS

_mesh = plsc.VectorSubcoreMesh(core_axis_name="c", subcore_axis_name="s")


def _scale_rows(rows, w_v, base, bias_v=None):
    def grp(g, c):
        wv = w_v[pl.ds(base + g * 16, 16)]
        for r in range(16):
            lane = jnp.full((16, 1), r, dtype=jnp.int32)
            ws = lax.gather(
                wv, lane,
                lax.GatherDimensionNumbers(
                    offset_dims=(), collapsed_slice_dims=(0,),
                    start_index_map=(0,)),
                slice_sizes=(1,),
                mode=lax.GatherScatterMode.PROMISE_IN_BOUNDS)
            row = g * 16 + r
            for j in range(8):
                sl = pl.ds(j * 16, 16)
                v = rows[row, sl]
                if bias_v is not None:
                    v = jnp.maximum(v, 0.0) + bias_v[j]
                rows[row, sl] = v * ws
        return c

    lax.fori_loop(0, K // 16, grp, 0)


def _sc_pass_body(src_hbm, gidx_hbm, sidx_hbm, w_hbm, part_out,
                  gidx_v, sidx_v, w_v, rows0, rows1, rows2, acc,
                  gsem0, gsem1, gsem2, ssem0, ssem1, ssem2, bias_ref=None):
    bias_v = None
    if bias_ref is not None:
        bias_v = tuple(bias_ref[pl.ds(j * 16, 16)] for j in range(8))
    cid = lax.axis_index("c")
    sid = lax.axis_index("s")
    wid = cid * NS + sid

    zv = jnp.zeros((16,), jnp.float32)

    def zrow(i, c):
        for j in range(8):
            rows0[i, pl.ds(j * 16, 16)] = zv
        return c

    lax.fori_loop(0, K, zrow, 0)
    zbase = sid * ROWS_PER_SUB
    for t in range(ROWS_PER_SUB // K):
        pltpu.sync_copy(rows0, acc.at[pl.ds(zbase + t * K, K)])
    plsc.subcore_barrier()

    def third(bl, cur, gsem_c, ssem_c, prv, gsem_p, ssem_p,
              first=False, last=False, guard=False):
        if first:
            @pl.when(bl > 0)
            def _():
                pltpu.make_async_copy(
                    prv, acc.at[sidx_v.at[bl - 1]], ssem_p).wait()
                pltpu.async_copy(src_hbm.at[gidx_v.at[bl + 2]], prv, gsem_p)

            @pl.when(bl == 0)
            def _():
                pltpu.async_copy(src_hbm.at[gidx_v.at[bl + 2]], prv, gsem_p)
        elif last:
            pltpu.make_async_copy(
                prv, acc.at[sidx_v.at[bl - 1]], ssem_p).wait()
        elif guard:
            pltpu.make_async_copy(
                prv, acc.at[sidx_v.at[bl - 1]], ssem_p).wait()

            @pl.when(bl + 2 < SB)
            def _():
                pltpu.async_copy(src_hbm.at[gidx_v.at[bl + 2]], prv, gsem_p)
        else:
            pltpu.make_async_copy(
                prv, acc.at[sidx_v.at[bl - 1]], ssem_p).wait()
            pltpu.async_copy(src_hbm.at[gidx_v.at[bl + 2]], prv, gsem_p)
        pltpu.make_async_copy(src_hbm.at[gidx_v.at[bl]], cur, gsem_c).wait()
        _scale_rows(cur, w_v, bl * K, bias_v)
        pltpu.async_copy(cur, acc.at[sidx_v.at[bl]], ssem_c, add=True)

    def chunk(s, carry):
        pltpu.sync_copy(gidx_hbm.at[wid, s], gidx_v)
        pltpu.sync_copy(sidx_hbm.at[wid, s], sidx_v)
        pltpu.sync_copy(w_hbm.at[pl.ds(wid * EPW + s * (SB * K), SB * K)],
                        w_v)
        pltpu.async_copy(src_hbm.at[gidx_v.at[0]], rows0, gsem0)
        pltpu.async_copy(src_hbm.at[gidx_v.at[1]], rows1, gsem1)

        def triple(t, c2):
            third(3 * t, rows0, gsem0, ssem0, rows2, gsem2, ssem2,
                  first=True)
            third(3 * t + 1, rows1, gsem1, ssem1, rows0, gsem0, ssem0)
            third(3 * t + 2, rows2, gsem2, ssem2, rows1, gsem1, ssem1,
                  guard=True)
            return c2

        lax.fori_loop(0, SB // 3, triple, 0)
        third(SB - 1, rows0, gsem0, ssem0, rows2, gsem2, ssem2, last=True)
        pltpu.make_async_copy(
            rows0, acc.at[sidx_v.at[SB - 1]], ssem0).wait()
        return carry

    lax.fori_loop(0, NSC, chunk, 0)
    plsc.subcore_barrier()

    pltpu.sync_copy(acc.at[pl.ds(zbase, ROWS_PER_SUB)],
                    part_out.at[cid, pl.ds(zbase, ROWS_PER_SUB)])


@functools.partial(
    pl.kernel,
    mesh=_mesh,
    out_type=[jax.ShapeDtypeStruct((2, NPAD, D), jnp.float32),
              jax.ShapeDtypeStruct((R, D), jnp.float32)],
    scratch_types=[
        pltpu.VMEM((SB, K), jnp.int32),
        pltpu.VMEM((SB, K), jnp.int32),
        pltpu.VMEM((SB * K,), jnp.float32),
        pltpu.VMEM((K, D), jnp.float32),
        pltpu.VMEM((K, D), jnp.float32),
        pltpu.VMEM((K, D), jnp.float32),
        pltpu.VMEM_SHARED((NPAD, D), jnp.float32),
        pltpu.SemaphoreType.DMA,
        pltpu.SemaphoreType.DMA,
        pltpu.SemaphoreType.DMA,
        pltpu.SemaphoreType.DMA,
        pltpu.SemaphoreType.DMA,
        pltpu.SemaphoreType.DMA,
        pltpu.VMEM((RPW,), jnp.int32),
        pltpu.VMEM((D,), jnp.float32),
        pltpu.SemaphoreType.DMA,
    ],
)
def _sc_pass1(x_hbm, gidx_hbm, sidx_hbm, w_hbm, bias_hbm, ridx_hbm,
              part_out, root_out,
              gidx_v, sidx_v, w_v, rows0, rows1, rows2, acc,
              gsem0, gsem1, gsem2, ssem0, ssem1, ssem2, ridx_v, bias_v,
              rsem):
    cid = lax.axis_index("c")
    sid = lax.axis_index("s")
    pltpu.sync_copy(bias_hbm, bias_v)

    @pl.when(cid == 0)
    def _root():
        pltpu.sync_copy(ridx_hbm.at[pl.ds(sid * RPW, RPW)], ridx_v)
        rdst = rows0.at[pl.ds(0, RPW)]
        pltpu.async_copy(x_hbm.at[ridx_v], rdst, rsem).wait()
        pltpu.sync_copy(rdst, root_out.at[pl.ds(sid * RPW, RPW)])

    _sc_pass_body(x_hbm, gidx_hbm, sidx_hbm, w_hbm, part_out,
                  gidx_v, sidx_v, w_v, rows0, rows1, rows2, acc,
                  gsem0, gsem1, gsem2, ssem0, ssem1, ssem2, bias_v)


@functools.partial(
    pl.kernel,
    mesh=_mesh,
    out_type=[jax.ShapeDtypeStruct((2, NPAD, D), jnp.float32)],
    scratch_types=[
        pltpu.VMEM((SB, K), jnp.int32),
        pltpu.VMEM((SB, K), jnp.int32),
        pltpu.VMEM((SB * K,), jnp.float32),
        pltpu.VMEM((K, D), jnp.float32),
        pltpu.VMEM((K, D), jnp.float32),
        pltpu.VMEM((K, D), jnp.float32),
        pltpu.VMEM_SHARED((NPAD, D), jnp.float32),
        pltpu.SemaphoreType.DMA,
        pltpu.SemaphoreType.DMA,
        pltpu.SemaphoreType.DMA,
        pltpu.SemaphoreType.DMA,
        pltpu.SemaphoreType.DMA,
        pltpu.SemaphoreType.DMA,
    ],
)
def _sc_pass2(src_hbm, gidx_hbm, sidx_hbm, w_hbm, part_out,
              gidx_v, sidx_v, w_v, rows0, rows1, rows2, acc,
              gsem0, gsem1, gsem2, ssem0, ssem1, ssem2):
    _sc_pass_body(src_hbm, gidx_hbm, sidx_hbm, w_hbm, part_out,
                  gidx_v, sidx_v, w_v, rows0, rows1, rows2, acc,
                  gsem0, gsem1, gsem2, ssem0, ssem1, ssem2)


_BLK = 400
_GRID = N // _BLK


def _sum_partials(part):
    def body(a_ref, b_ref, o_ref):
        o_ref[...] = a_ref[0] + b_ref[0]

    return pl.pallas_call(
        body,
        grid=(_GRID,),
        in_specs=[pl.BlockSpec((1, _BLK, D), lambda i: (0, i, 0)),
                  pl.BlockSpec((1, _BLK, D), lambda i: (1, i, 0))],
        out_specs=pl.BlockSpec((_BLK, D), lambda i: (i, 0)),
        out_shape=jax.ShapeDtypeStruct((N, D), jnp.float32),
    )(part, part)


def _combine(part, x, fc1_Wt, W1t, b1, w2, b2):
    def body(p0_ref, p1_ref, x_ref, fw_ref, w1_ref, b1_ref, w2_ref, b2_ref,
             o_ref):
        node = p0_ref[0] + p1_ref[0]
        node = node - jnp.max(node, axis=1, keepdims=True)
        ex = jnp.exp(node)
        sm = ex / jnp.sum(ex, axis=1, keepdims=True)
        node2 = jnp.dot(sm, fw_ref[...], preferred_element_type=jnp.float32)
        xb = x_ref[...]
        b1v = b1_ref[...]
        tx = jnp.tanh(jnp.dot(xb, w1_ref[...],
                              preferred_element_type=jnp.float32) + b1v)
        tn = jnp.tanh(jnp.dot(node2, w1_ref[...],
                              preferred_element_type=jnp.float32) + b1v)
        w2v = w2_ref[...]
        b2v = b2_ref[0, 0]
        sx = jnp.sum(tx * w2v, axis=1, keepdims=True) + b2v
        sn = jnp.sum(tn * w2v, axis=1, keepdims=True) + b2v
        m = jnp.maximum(sx, sn)
        ax = jnp.exp(sx - m)
        an = jnp.exp(sn - m)
        o_ref[...] = (ax * xb + an * node2) / (ax + an)

    return pl.pallas_call(
        body,
        grid=(_GRID,),
        in_specs=[pl.BlockSpec((1, _BLK, D), lambda i: (0, i, 0)),
                  pl.BlockSpec((1, _BLK, D), lambda i: (1, i, 0)),
                  pl.BlockSpec((_BLK, D), lambda i: (i, 0)),
                  pl.BlockSpec((D, D), lambda i: (0, 0)),
                  pl.BlockSpec((D, D), lambda i: (0, 0)),
                  pl.BlockSpec((1, D), lambda i: (0, 0)),
                  pl.BlockSpec((1, D), lambda i: (0, 0)),
                  pl.BlockSpec((1, 1), lambda i: (0, 0))],
        out_specs=pl.BlockSpec((_BLK, D), lambda i: (i, 0)),
        out_shape=jax.ShapeDtypeStruct((N, D), jnp.float32),
    )(part, part, x, fc1_Wt, W1t, b1, w2, b2)


def kernel(x, edge_index, edge_weight, root_idx, hgc1_bias, fc1_W,
           fus_l1_W, fus_l1_b, fus_l2_W, fus_l2_b):
    row = edge_index[0].reshape(NW, NSC, SB, K)
    col = edge_index[1].reshape(NW, NSC, SB, K)

    part1, root_emb = _sc_pass1(x, row, col, edge_weight, hgc1_bias,
                                root_idx)
    edge = _sum_partials(part1)
    (part2,) = _sc_pass2(edge, col, row, edge_weight)
    out = _combine(part2, x, fc1_W.T, fus_l1_W.T,
                   fus_l1_b.reshape(1, D), fus_l2_W.reshape(1, D),
                   fus_l2_b.reshape(1, 1))
    return (out, edge, root_emb)


# speedup vs baseline: 2.2579x; 1.0645x over previous
import functools

import jax
import jax.numpy as jnp
from jax import lax
from jax.experimental import pallas as pl
from jax.experimental.pallas import tpu as pltpu
from jax.experimental.pallas import tpu_sc as plsc

N = 10000
D = 128
E = 320000
R = 512

NC = 2
NS = 16
NW = NC * NS
EPW = E // NW
K = 80
NB = EPW // K
SB = 25
NSC = NB // SB
NPAD = 10240
ROWS_PER_SUB = NPAD // NS
RPW = R // NS

_mesh = plsc.VectorSubcoreMesh(core_axis_name="c", subcore_axis_name="s")


def _scale_rows(rows, w_v, base, bias_v=None):
    def grp(g, c):
        wv = w_v[pl.ds(base + g * 16, 16)]
        for r in range(16):
            lane = jnp.full((16, 1), r, dtype=jnp.int32)
            ws = lax.gather(
                wv, lane,
                lax.GatherDimensionNumbers(
                    offset_dims=(), collapsed_slice_dims=(0,),
                    start_index_map=(0,)),
                slice_sizes=(1,),
                mode=lax.GatherScatterMode.PROMISE_IN_BOUNDS)
            row = g * 16 + r
            for j in range(8):
                sl = pl.ds(j * 16, 16)
                v = rows[row, sl]
                if bias_v is not None:
                    v = jnp.maximum(v, 0.0) + bias_v[j]
                rows[row, sl] = v * ws
        return c

    lax.fori_loop(0, K // 16, grp, 0)


def _sc_pass_body(src_hbm, gidx_hbm, sidx_hbm, w_hbm, part_out,
                  gidx_v, sidx_v, w_v, rows0, rows1, rows2, acc,
                  gsem0, gsem1, gsem2, ssem0, ssem1, ssem2, bias_ref=None):
    bias_v = None
    if bias_ref is not None:
        bias_v = tuple(bias_ref[pl.ds(j * 16, 16)] for j in range(8))
    cid = lax.axis_index("c")
    sid = lax.axis_index("s")
    wid = cid * NS + sid

    zv = jnp.zeros((16,), jnp.float32)

    def zrow(i, c):
        for j in range(8):
            rows0[i, pl.ds(j * 16, 16)] = zv
        return c

    lax.fori_loop(0, K, zrow, 0)
    zbase = sid * ROWS_PER_SUB
    for t in range(ROWS_PER_SUB // K):
        pltpu.sync_copy(rows0, acc.at[pl.ds(zbase + t * K, K)])
    plsc.subcore_barrier()

    def issue_gather(bl2, dst, sem):
        pltpu.async_copy(src_hbm.at[gidx_v.at[bl2, pl.ds(0, K // 2)]],
                         dst.at[pl.ds(0, K // 2)], sem)
        pltpu.async_copy(src_hbm.at[gidx_v.at[bl2, pl.ds(K // 2, K // 2)]],
                         dst.at[pl.ds(K // 2, K // 2)], sem)

    def third(bl, cur, gsem_c, ssem_c, prv, gsem_p, ssem_p,
              first=False, last=False, guard=False):
        if first:
            @pl.when(bl > 0)
            def _():
                pltpu.make_async_copy(
                    prv, acc.at[sidx_v.at[bl - 1]], ssem_p).wait()
                issue_gather(bl + 2, prv, gsem_p)

            @pl.when(bl == 0)
            def _():
                issue_gather(bl + 2, prv, gsem_p)
        elif last:
            pltpu.make_async_copy(
                prv, acc.at[sidx_v.at[bl - 1]], ssem_p).wait()
        elif guard:
            pltpu.make_async_copy(
                prv, acc.at[sidx_v.at[bl - 1]], ssem_p).wait()

            @pl.when(bl + 2 < SB)
            def _():
                issue_gather(bl + 2, prv, gsem_p)
        else:
            pltpu.make_async_copy(
                prv, acc.at[sidx_v.at[bl - 1]], ssem_p).wait()
            issue_gather(bl + 2, prv, gsem_p)
        pltpu.make_async_copy(src_hbm.at[gidx_v.at[bl]], cur, gsem_c).wait()
        _scale_rows(cur, w_v, bl * K, bias_v)
        pltpu.async_copy(cur, acc.at[sidx_v.at[bl]], ssem_c, add=True)

    def chunk(s, carry):
        pltpu.sync_copy(gidx_hbm.at[wid, s], gidx_v)
        pltpu.sync_copy(sidx_hbm.at[wid, s], sidx_v)
        pltpu.sync_copy(w_hbm.at[pl.ds(wid * EPW + s * (SB * K), SB * K)],
                        w_v)
        issue_gather(0, rows0, gsem0)
        issue_gather(1, rows1, gsem1)

        def triple(t, c2):
            third(3 * t, rows0, gsem0, ssem0, rows2, gsem2, ssem2,
                  first=True)
            third(3 * t + 1, rows1, gsem1, ssem1, rows0, gsem0, ssem0)
            third(3 * t + 2, rows2, gsem2, ssem2, rows1, gsem1, ssem1,
                  guard=True)
            return c2

        lax.fori_loop(0, SB // 3, triple, 0)
        third(SB - 1, rows0, gsem0, ssem0, rows2, gsem2, ssem2, last=True)
        pltpu.make_async_copy(
            rows0, acc.at[sidx_v.at[SB - 1]], ssem0).wait()
        return carry

    lax.fori_loop(0, NSC, chunk, 0)
    plsc.subcore_barrier()

    pltpu.sync_copy(acc.at[pl.ds(zbase, ROWS_PER_SUB)],
                    part_out.at[cid, pl.ds(zbase, ROWS_PER_SUB)])


@functools.partial(
    pl.kernel,
    mesh=_mesh,
    out_type=[jax.ShapeDtypeStruct((2, NPAD, D), jnp.float32),
              jax.ShapeDtypeStruct((R, D), jnp.float32)],
    scratch_types=[
        pltpu.VMEM((SB, K), jnp.int32),
        pltpu.VMEM((SB, K), jnp.int32),
        pltpu.VMEM((SB * K,), jnp.float32),
        pltpu.VMEM((K, D), jnp.float32),
        pltpu.VMEM((K, D), jnp.float32),
        pltpu.VMEM((K, D), jnp.float32),
        pltpu.VMEM_SHARED((NPAD, D), jnp.float32),
        pltpu.SemaphoreType.DMA,
        pltpu.SemaphoreType.DMA,
        pltpu.SemaphoreType.DMA,
        pltpu.SemaphoreType.DMA,
        pltpu.SemaphoreType.DMA,
        pltpu.SemaphoreType.DMA,
        pltpu.VMEM((RPW,), jnp.int32),
        pltpu.VMEM((D,), jnp.float32),
        pltpu.SemaphoreType.DMA,
    ],
)
def _sc_pass1(x_hbm, gidx_hbm, sidx_hbm, w_hbm, bias_hbm, ridx_hbm,
              part_out, root_out,
              gidx_v, sidx_v, w_v, rows0, rows1, rows2, acc,
              gsem0, gsem1, gsem2, ssem0, ssem1, ssem2, ridx_v, bias_v,
              rsem):
    cid = lax.axis_index("c")
    sid = lax.axis_index("s")
    pltpu.sync_copy(bias_hbm, bias_v)

    @pl.when(cid == 0)
    def _root():
        pltpu.sync_copy(ridx_hbm.at[pl.ds(sid * RPW, RPW)], ridx_v)
        rdst = rows0.at[pl.ds(0, RPW)]
        pltpu.async_copy(x_hbm.at[ridx_v], rdst, rsem).wait()
        pltpu.sync_copy(rdst, root_out.at[pl.ds(sid * RPW, RPW)])

    _sc_pass_body(x_hbm, gidx_hbm, sidx_hbm, w_hbm, part_out,
                  gidx_v, sidx_v, w_v, rows0, rows1, rows2, acc,
                  gsem0, gsem1, gsem2, ssem0, ssem1, ssem2, bias_v)


@functools.partial(
    pl.kernel,
    mesh=_mesh,
    out_type=[jax.ShapeDtypeStruct((2, NPAD, D), jnp.float32)],
    scratch_types=[
        pltpu.VMEM((SB, K), jnp.int32),
        pltpu.VMEM((SB, K), jnp.int32),
        pltpu.VMEM((SB * K,), jnp.float32),
        pltpu.VMEM((K, D), jnp.float32),
        pltpu.VMEM((K, D), jnp.float32),
        pltpu.VMEM((K, D), jnp.float32),
        pltpu.VMEM_SHARED((NPAD, D), jnp.float32),
        pltpu.SemaphoreType.DMA,
        pltpu.SemaphoreType.DMA,
        pltpu.SemaphoreType.DMA,
        pltpu.SemaphoreType.DMA,
        pltpu.SemaphoreType.DMA,
        pltpu.SemaphoreType.DMA,
    ],
)
def _sc_pass2(src_hbm, gidx_hbm, sidx_hbm, w_hbm, part_out,
              gidx_v, sidx_v, w_v, rows0, rows1, rows2, acc,
              gsem0, gsem1, gsem2, ssem0, ssem1, ssem2):
    _sc_pass_body(src_hbm, gidx_hbm, sidx_hbm, w_hbm, part_out,
                  gidx_v, sidx_v, w_v, rows0, rows1, rows2, acc,
                  gsem0, gsem1, gsem2, ssem0, ssem1, ssem2)


_BLK = 2000
_GRID = N // _BLK


def _sum_partials(part):
    def body(a_ref, b_ref, o_ref):
        o_ref[...] = a_ref[0] + b_ref[0]

    return pl.pallas_call(
        body,
        grid=(_GRID,),
        in_specs=[pl.BlockSpec((1, _BLK, D), lambda i: (0, i, 0)),
                  pl.BlockSpec((1, _BLK, D), lambda i: (1, i, 0))],
        out_specs=pl.BlockSpec((_BLK, D), lambda i: (i, 0)),
        out_shape=jax.ShapeDtypeStruct((N, D), jnp.float32),
    )(part, part)


def _combine(part, x, fc1_Wt, W1t, b1, w2, b2):
    def body(p0_ref, p1_ref, x_ref, fw_ref, w1_ref, b1_ref, w2_ref, b2_ref,
             o_ref):
        node = p0_ref[0] + p1_ref[0]
        node = node - jnp.max(node, axis=1, keepdims=True)
        ex = jnp.exp(node)
        sm = ex / jnp.sum(ex, axis=1, keepdims=True)
        node2 = jnp.dot(sm, fw_ref[...], preferred_element_type=jnp.float32)
        xb = x_ref[...]
        b1v = b1_ref[...]
        tx = jnp.tanh(jnp.dot(xb, w1_ref[...],
                              preferred_element_type=jnp.float32) + b1v)
        tn = jnp.tanh(jnp.dot(node2, w1_ref[...],
                              preferred_element_type=jnp.float32) + b1v)
        w2v = w2_ref[...]
        b2v = b2_ref[0, 0]
        sx = jnp.sum(tx * w2v, axis=1, keepdims=True) + b2v
        sn = jnp.sum(tn * w2v, axis=1, keepdims=True) + b2v
        m = jnp.maximum(sx, sn)
        ax = jnp.exp(sx - m)
        an = jnp.exp(sn - m)
        o_ref[...] = (ax * xb + an * node2) / (ax + an)

    return pl.pallas_call(
        body,
        grid=(_GRID,),
        in_specs=[pl.BlockSpec((1, _BLK, D), lambda i: (0, i, 0)),
                  pl.BlockSpec((1, _BLK, D), lambda i: (1, i, 0)),
                  pl.BlockSpec((_BLK, D), lambda i: (i, 0)),
                  pl.BlockSpec((D, D), lambda i: (0, 0)),
                  pl.BlockSpec((D, D), lambda i: (0, 0)),
                  pl.BlockSpec((1, D), lambda i: (0, 0)),
                  pl.BlockSpec((1, D), lambda i: (0, 0)),
                  pl.BlockSpec((1, 1), lambda i: (0, 0))],
        out_specs=pl.BlockSpec((_BLK, D), lambda i: (i, 0)),
        out_shape=jax.ShapeDtypeStruct((N, D), jnp.float32),
    )(part, part, x, fc1_Wt, W1t, b1, w2, b2)


def kernel(x, edge_index, edge_weight, root_idx, hgc1_bias, fc1_W,
           fus_l1_W, fus_l1_b, fus_l2_W, fus_l2_b):
    row = edge_index[0].reshape(NW, NSC, SB, K)
    col = edge_index[1].reshape(NW, NSC, SB, K)

    part1, root_emb = _sc_pass1(x, row, col, edge_weight, hgc1_bias,
                                root_idx)
    edge = _sum_partials(part1)
    (part2,) = _sc_pass2(edge, col, row, edge_weight)
    out = _combine(part2, x, fc1_W.T, fus_l1_W.T,
                   fus_l1_b.reshape(1, D), fus_l2_W.reshape(1, D),
                   fus_l2_b.reshape(1, 1))
    return (out, edge, root_emb)
